# baseline (speedup 1.0000x reference)
import functools

import jax
import jax.numpy as jnp
from jax import lax
from jax.experimental import pallas as pl
from jax.experimental.pallas import tpu as pltpu
from jax.experimental.pallas import tpu_sc as plsc

N = 50000
E = 800000
B = 512
L = 1500
V = 26
D = 32

NW = 32
NT = 16
CHUNK = 128
G = 1
SGS = 8
NSG = 25
NG = SGS * NSG
CPT = NSG * SGS * G
E_PAD = NW * CPT * CHUNK
RACC = 50048
R = 53248
BACC = 640
PCH = R // NW // CHUNK


def _sc_mesh():
    return plsc.VectorSubcoreMesh(core_axis_name="c", subcore_axis_name="s")


def _edge_segsum(y, srcp, dstp, zeros):
    def body(y_hbm, src_hbm, dst_hbm, zero_hbm, out_hbm,
             srci_v, dsti_v, ring_v, acc_sh, sem_i, sem_g, sem_s):
        c = lax.axis_index("c")
        s = lax.axis_index("s")
        wid = c * NT + s
        tpt = RACC // NT

        def idx_load(sg, slot):
            return [pltpu.make_async_copy(src_hbm.at[wid, sg],
                                          srci_v.at[slot], sem_i),
                    pltpu.make_async_copy(dst_hbm.at[wid, sg],
                                          dsti_v.at[slot], sem_i)]


        def gather(half, slot, row):
            return pltpu.make_async_copy(y_hbm.at[srci_v.at[slot, row]],
                                         ring_v.at[half], sem_g)

        def scatter(half, slot, row):
            return pltpu.make_async_copy(ring_v.at[half],
                                         acc_sh.at[dsti_v.at[slot, row]],
                                         sem_s)

        def gpos(g):
            return (g % 2,
                    (g // SGS) % 2,
                    g % SGS)

        pltpu.sync_copy(zero_hbm, acc_sh.at[pl.ds(s * tpt, tpt)])
        for dd in idx_load(0, 0):
            dd.start()
            dd.wait()
        gather(0, 0, 0).start()
        for dd in idx_load(1, 1):
            dd.start()
        plsc.subcore_barrier()

        def grp_body(g, carry):
            h, sl, row = gpos(g)
            gather(h, sl, row).wait()
            scatter(h, sl, row).start(add=True)

            @pl.when(g >= 1)
            def _():
                hp, slp, rowp = gpos(g - 1)
                scatter(hp, slp, rowp).wait()

            @pl.when(g <= NG - 2)
            def _():
                hn, sln, rown = gpos(g + 1)

                @pl.when(rown == 0)
                def _():
                    for dd in idx_load((g + 1) // SGS, sln):
                        dd.wait()

                gather(hn, sln, rown).start()

            @pl.when((g % SGS == 1) & (g >= SGS + 1)
                     & (g <= SGS * (NSG - 2) + 1))
            def _():
                sgn = g // SGS + 1
                for dd in idx_load(sgn, sgn % 2):
                    dd.start()

            return carry

        lax.fori_loop(0, NG, grp_body, 0)
        hl, sll, rowl = gpos(NG - 1)
        scatter(hl, sll, rowl).wait()
        plsc.subcore_barrier()
        pltpu.sync_copy(acc_sh.at[pl.ds(s * tpt, tpt)],
                        out_hbm.at[c, pl.ds(s * tpt, tpt)])

    f = pl.kernel(
        body,
        out_type=jax.ShapeDtypeStruct((2, RACC, D), jnp.float32),
        mesh=_sc_mesh(),
        compiler_params=pltpu.CompilerParams(use_tc_tiling_on_sc=False),
        scratch_types=[
            pltpu.VMEM((2, SGS, G * CHUNK), jnp.int32),
            pltpu.VMEM((2, SGS, G * CHUNK), jnp.int32),
            pltpu.VMEM((2, G * CHUNK, D), jnp.float32),
            pltpu.VMEM_SHARED((RACC, D), jnp.float32),
            pltpu.SemaphoreType.DMA,
            pltpu.SemaphoreType.DMA,
            pltpu.SemaphoreType.DMA,
        ],
    )
    return f(y, srcp, dstp, zeros)


def _pool_segsum(h5, batchp, zeros):
    def body(h_hbm, b_hbm, zero_hbm, out_hbm, hbuf_v, bidx_v, acc_sh, sem_s):
        c = lax.axis_index("c")
        s = lax.axis_index("s")
        wid = c * NT + s
        rows = PCH * CHUNK
        pltpu.sync_copy(h_hbm.at[pl.ds(wid * rows, rows)], hbuf_v)
        pltpu.sync_copy(b_hbm.at[pl.ds(wid * PCH, PCH)], bidx_v)
        pltpu.sync_copy(zero_hbm.at[pl.ds(0, BACC // NT)],
                        acc_sh.at[pl.ds(s * (BACC // NT), BACC // NT)])
        plsc.subcore_barrier()
        sds = [pltpu.make_async_copy(hbuf_v.at[pl.ds(j * CHUNK, CHUNK)],
                                     acc_sh.at[bidx_v.at[j]], sem_s)
               for j in range(PCH)]
        for d in sds:
            d.start(add=True)
        for d in sds:
            d.wait()
        plsc.subcore_barrier()
        pltpu.sync_copy(acc_sh.at[pl.ds(s * (BACC // NT), BACC // NT)],
                        out_hbm.at[c, pl.ds(s * (BACC // NT), BACC // NT)])

    f = pl.kernel(
        body,
        out_type=jax.ShapeDtypeStruct((2, BACC, D), jnp.float32),
        mesh=_sc_mesh(),
        compiler_params=pltpu.CompilerParams(use_tc_tiling_on_sc=False),
        scratch_types=[
            pltpu.VMEM((PCH * CHUNK, D), jnp.float32),
            pltpu.VMEM((PCH, CHUNK), jnp.int32),
            pltpu.VMEM_SHARED((BACC, D), jnp.float32),
            pltpu.SemaphoreType.DMA,
        ],
    )
    return f(h5, batchp, zeros)


def _mm_body(x_ref, w_ref, o_ref):
    o_ref[...] = jnp.dot(x_ref[...], w_ref[...],
                         preferred_element_type=jnp.float32)


def _tc_y0(x4, w_bd):
    blk = 512
    return pl.pallas_call(
        _mm_body,
        grid=(pl.cdiv(N // 4, blk),),
        in_specs=[pl.BlockSpec((blk, 4 * 78), lambda i: (i, 0)),
                  pl.BlockSpec((4 * 78, 4 * D), lambda i: (0, 0))],
        out_specs=pl.BlockSpec((blk, 4 * D), lambda i: (i, 0)),
        out_shape=jax.ShapeDtypeStruct((N // 4, 4 * D), jnp.float32),
    )(x4, w_bd)


def _combine_body(y_ref, s_ref, wb_ref, wa_ref, vec_ref, o_ref):
    vec = vec_ref[...]
    sp = s_ref[...]
    t = jnp.maximum(y_ref[...] + sp[0] + sp[1] + vec[0:1, :], 0.0)
    z = jnp.maximum(jnp.dot(t, wb_ref[...],
                            preferred_element_type=jnp.float32)
                    + vec[1:2, :], 0.0)
    h = z * vec[2:3, :] + vec[3:4, :]
    o_ref[...] = jnp.dot(h, wa_ref[...], preferred_element_type=jnp.float32)


def _tc_combine(y, s_part, wb_bd, wa_bd, vecs4):
    blk = 1664
    dd = 4 * D
    return pl.pallas_call(
        _combine_body,
        grid=(pl.cdiv(N // 4, blk),),
        in_specs=[pl.BlockSpec((blk, dd), lambda i: (i, 0)),
                  pl.BlockSpec((2, blk, dd), lambda i: (0, i, 0)),
                  pl.BlockSpec((dd, dd), lambda i: (0, 0)),
                  pl.BlockSpec((dd, dd), lambda i: (0, 0)),
                  pl.BlockSpec((4, dd), lambda i: (0, 0))],
        out_specs=pl.BlockSpec((blk, dd), lambda i: (i, 0)),
        out_shape=jax.ShapeDtypeStruct((N // 4, dd), jnp.float32),
    )(y, s_part, wb_bd, wa_bd, vecs4)


def _final_layer_body(y_ref, s_ref, wb_ref, vec_ref, o_ref):
    vec = vec_ref[...]
    sp = s_ref[...]
    t = jnp.maximum(y_ref[...] + sp[0] + sp[1] + vec[0:1, :], 0.0)
    z = jnp.maximum(jnp.dot(t, wb_ref[...],
                            preferred_element_type=jnp.float32)
                    + vec[1:2, :], 0.0)
    o_ref[...] = z * vec[2:3, :] + vec[3:4, :]


def _tc_final_layer(y, s_part, wb_bd, vecs4):
    blk = 1664
    dd = 4 * D
    return pl.pallas_call(
        _final_layer_body,
        grid=(pl.cdiv(N // 4, blk),),
        in_specs=[pl.BlockSpec((blk, dd), lambda i: (i, 0)),
                  pl.BlockSpec((2, blk, dd), lambda i: (0, i, 0)),
                  pl.BlockSpec((dd, dd), lambda i: (0, 0)),
                  pl.BlockSpec((4, dd), lambda i: (0, 0))],
        out_specs=pl.BlockSpec((blk, dd), lambda i: (i, 0)),
        out_shape=jax.ShapeDtypeStruct((R // 4, dd), jnp.float32),
    )(y, s_part, wb_bd, vecs4)


def _q_body(t_ref, w_ref, o_ref):
    v = pl.program_id(0)
    mask = (t_ref[...] == v).astype(jnp.float32)
    o_ref[...] = jnp.dot(mask, w_ref[...],
                         preferred_element_type=jnp.float32)[None]


def _tc_q(target, wp):
    return pl.pallas_call(
        _q_body,
        grid=(V,),
        in_specs=[pl.BlockSpec((B, L), lambda v: (0, 0)),
                  pl.BlockSpec((L, 256), lambda v: (0, 0))],
        out_specs=pl.BlockSpec((1, B, 256), lambda v: (v, 0, 0)),
        out_shape=jax.ShapeDtypeStruct((V, B, 256), jnp.float32),
    )(target, wp)


def _conv2_body(q_ref, e_ref, cb_ref, o_ref):
    o_ref[...] = (jnp.dot(q_ref[...], e_ref[...],
                          preferred_element_type=jnp.float32)
                  + cb_ref[...])


def _tc_conv2(qbig, ehat, cbcol):
    blk = 2048
    return pl.pallas_call(
        _conv2_body,
        grid=(B * D // blk,),
        in_specs=[pl.BlockSpec((blk, V * 8), lambda i: (i, 0)),
                  pl.BlockSpec((V * 8, 121), lambda i: (0, 0)),
                  pl.BlockSpec((blk, 1), lambda i: (0, 0))],
        out_specs=pl.BlockSpec((blk, 121), lambda i: (i, 0)),
        out_shape=jax.ShapeDtypeStruct((B * D, 121), jnp.float32),
    )(qbig, ehat, cbcol)


def _xt_body(c_ref, w_ref, b_ref, o_ref):
    o_ref[...] = (jnp.dot(c_ref[...], w_ref[...],
                          preferred_element_type=jnp.float32)
                  + b_ref[...])


def _tc_xt(conv3, w2, bias_eff):
    return pl.pallas_call(
        _xt_body,
        out_shape=jax.ShapeDtypeStruct((B, 128), jnp.float32),
    )(conv3, w2, bias_eff)


def _head_body(p_ref, xt_ref, wxd_ref, bxd_ref, f1a_ref, f1b_ref, b1_ref,
               w2_ref, b2_ref, wo_ref, bo_ref, o_ref):
    pall = p_ref[...]
    pooled = pall[0, :B, :] + pall[1, :B, :]
    xd = jnp.maximum(jnp.dot(pooled, wxd_ref[...],
                             preferred_element_type=jnp.float32)
                     + bxd_ref[...], 0.0)
    xc = jnp.maximum(jnp.dot(xd, f1a_ref[...],
                             preferred_element_type=jnp.float32)
                     + jnp.dot(xt_ref[...], f1b_ref[...],
                               preferred_element_type=jnp.float32)
                     + b1_ref[...], 0.0)
    x2 = jnp.maximum(jnp.dot(xc, w2_ref[...],
                             preferred_element_type=jnp.float32)
                     + b2_ref[...], 0.0)
    o_ref[...] = (jnp.dot(x2, wo_ref[...],
                          preferred_element_type=jnp.float32)
                  + bo_ref[...])


def _tc_head(pooled_part, xt, p):
    args = (pooled_part, xt,
            p['fc1_xd_W'], p['fc1_xd_b'].reshape(1, 128),
            p['fc1_W'][:128], p['fc1_W'][128:], p['fc1_b'].reshape(1, 1024),
            p['fc2_W'], p['fc2_b'].reshape(1, 256),
            p['out_W'], p['out_b'].reshape(1, 1))
    return pl.pallas_call(
        _head_body,
        out_shape=jax.ShapeDtypeStruct((B, 1), jnp.float32),
    )(*args)


def kernel(x, edge_index, batch, target, params):
    p = params
    f32 = jnp.float32

    pad_i = jnp.arange(E_PAD - E, dtype=jnp.int32)
    srcp = jnp.concatenate([edge_index[0], pad_i % N]
                           ).reshape(NW, NSG, SGS, G * CHUNK)
    dstp = jnp.concatenate([edge_index[1], N + pad_i % (RACC - N)]
                           ).reshape(NW, NSG, SGS, G * CHUNK)
    pad_b = jnp.arange(R - N, dtype=jnp.int32)
    batchp = jnp.concatenate([batch, B + pad_b % (BACC - B)]
                             ).reshape(NW * PCH, CHUNK)
    zeros = jnp.zeros((RACC // NT, D), f32)

    eye4 = jnp.eye(4, dtype=f32)
    vecs4 = [jnp.stack([jnp.tile(p['b%da' % i], 4),
                        jnp.tile(p['b%db' % i], 4),
                        jnp.tile(p['bn%d_g' % i] / jnp.sqrt(f32(1.0 + 1e-5)),
                                 4),
                        jnp.tile(p['bn%d_b' % i], 4)]) for i in range(5)]
    wa_bd = [jnp.kron(eye4, p['W%da' % i]) for i in range(5)]
    wb_bd = [jnp.kron(eye4, p['W%db' % i]) for i in range(5)]

    x4 = x.reshape(N // 4, 4 * 78)
    y = _tc_y0(x4, jnp.kron(eye4, p['W0a']))
    s_part = _edge_segsum(y.reshape(N, D), srcp, dstp, zeros)

    wp = p['conv_W'].transpose(1, 0, 2).reshape(L, 256)
    q = _tc_q(target, wp)
    qbig = q.reshape(V, B, D, 8).transpose(1, 2, 0, 3).reshape(B * D, V * 8)
    ehat = jnp.stack([p['emb'][:, k:k + 121] for k in range(8)],
                     axis=1).reshape(V * 8, 121)
    cbcol = jnp.tile(p['conv_b'], 2048 // D).reshape(2048, 1)
    convflat = _tc_conv2(qbig, ehat, cbcol)
    conv3 = convflat.reshape(B, D * 121)
    xt = _tc_xt(conv3, p['fc1_xt_W'], p['fc1_xt_b'].reshape(1, 128))

    for i in range(4):
        y = _tc_combine(y, s_part.reshape(2, RACC // 4, 4 * D),
                        wb_bd[i], wa_bd[i + 1], vecs4[i])
        s_part = _edge_segsum(y.reshape(N, D), srcp, dstp, zeros)
    h5 = _tc_final_layer(y, s_part.reshape(2, RACC // 4, 4 * D),
                         wb_bd[4], vecs4[4])
    pooled_part = _pool_segsum(h5.reshape(R, D), batchp, zeros)

    return _tc_head(pooled_part, xt, p)


# speedup vs baseline: 1.4387x; 1.4387x over previous
import functools

import jax
import jax.numpy as jnp
from jax import lax
from jax.experimental import pallas as pl
from jax.experimental.pallas import tpu as pltpu
from jax.experimental.pallas import tpu_sc as plsc

N = 50000
E = 800000
B = 512
L = 1500
V = 26
D = 32

NW = 32
NT = 16
CHUNK = 128
G = 1
SGS = 8
NSG = 25
NG = SGS * NSG
CPT = NSG * SGS * G
E_PAD = NW * CPT * CHUNK
RACC = 50048
R = 53248
BACC = 640
PCH = R // NW // CHUNK


def _sc_mesh():
    return plsc.VectorSubcoreMesh(core_axis_name="c", subcore_axis_name="s")


def _edge_segsum(y, srcp, dstp, zeros):
    def body(y_hbm, src_hbm, dst_hbm, zero_hbm, out_hbm,
             srci_v, dsti_v, ring_v, acc_sh, sem_i, sem_g, sem_s):
        c = lax.axis_index("c")
        s = lax.axis_index("s")
        wid = c * NT + s
        tpt = RACC // NT

        def idx_load(sg, slot):
            return [pltpu.make_async_copy(src_hbm.at[wid, sg],
                                          srci_v.at[slot], sem_i),
                    pltpu.make_async_copy(dst_hbm.at[wid, sg],
                                          dsti_v.at[slot], sem_i)]


        def gather(half, slot, row):
            return pltpu.make_async_copy(y_hbm.at[srci_v.at[slot, row]],
                                         ring_v.at[half], sem_g)

        def scatter(half, slot, row):
            return pltpu.make_async_copy(ring_v.at[half],
                                         acc_sh.at[dsti_v.at[slot, row]],
                                         sem_s)

        def gpos(g):
            return (g % 4,
                    (g // SGS) % 3,
                    g % SGS)

        pltpu.sync_copy(zero_hbm, acc_sh.at[pl.ds(s * tpt, tpt)])
        for dd in idx_load(0, 0):
            dd.start()
            dd.wait()
        gather(0, 0, 0).start()
        gather(1, 0, 1).start()
        for dd in idx_load(1, 1):
            dd.start()
        for dd in idx_load(2, 2):
            dd.start()
        plsc.subcore_barrier()

        def grp_body(g, carry):
            h, sl, row = gpos(g)
            gather(h, sl, row).wait()
            scatter(h, sl, row).start(add=True)

            @pl.when(g >= 2)
            def _():
                hp, slp, rowp = gpos(g - 2)
                scatter(hp, slp, rowp).wait()

            @pl.when(g <= NG - 3)
            def _():
                hn, sln, rown = gpos(g + 2)

                @pl.when(rown == 0)
                def _():
                    for dd in idx_load((g + 2) // SGS, sln):
                        dd.wait()

                gather(hn, sln, rown).start()

            @pl.when((g % SGS == 2) & (g >= SGS + 2)
                     & (g <= SGS * (NSG - 3) + 2))
            def _():
                sgn = g // SGS + 2
                for dd in idx_load(sgn, sgn % 3):
                    dd.start()

            return carry

        lax.fori_loop(0, NG, grp_body, 0)
        for gl in (NG - 2, NG - 1):
            hl, sll, rowl = gpos(gl)
            scatter(hl, sll, rowl).wait()
        plsc.subcore_barrier()
        pltpu.sync_copy(acc_sh.at[pl.ds(s * tpt, tpt)],
                        out_hbm.at[c, pl.ds(s * tpt, tpt)])

    f = pl.kernel(
        body,
        out_type=jax.ShapeDtypeStruct((2, RACC, D), jnp.float32),
        mesh=_sc_mesh(),
        compiler_params=pltpu.CompilerParams(use_tc_tiling_on_sc=False),
        scratch_types=[
            pltpu.VMEM((3, SGS, G * CHUNK), jnp.int32),
            pltpu.VMEM((3, SGS, G * CHUNK), jnp.int32),
            pltpu.VMEM((4, G * CHUNK, D), jnp.float32),
            pltpu.VMEM_SHARED((RACC, D), jnp.float32),
            pltpu.SemaphoreType.DMA,
            pltpu.SemaphoreType.DMA,
            pltpu.SemaphoreType.DMA,
        ],
    )
    return f(y, srcp, dstp, zeros)


def _pool_segsum(h5, batchp, zeros):
    def body(h_hbm, b_hbm, zero_hbm, out_hbm, hbuf_v, bidx_v, acc_sh, sem_s):
        c = lax.axis_index("c")
        s = lax.axis_index("s")
        wid = c * NT + s
        rows = PCH * CHUNK
        pltpu.sync_copy(h_hbm.at[pl.ds(wid * rows, rows)], hbuf_v)
        pltpu.sync_copy(b_hbm.at[pl.ds(wid * PCH, PCH)], bidx_v)
        pltpu.sync_copy(zero_hbm.at[pl.ds(0, BACC // NT)],
                        acc_sh.at[pl.ds(s * (BACC // NT), BACC // NT)])
        plsc.subcore_barrier()
        sds = [pltpu.make_async_copy(hbuf_v.at[pl.ds(j * CHUNK, CHUNK)],
                                     acc_sh.at[bidx_v.at[j]], sem_s)
               for j in range(PCH)]
        for d in sds:
            d.start(add=True)
        for d in sds:
            d.wait()
        plsc.subcore_barrier()
        pltpu.sync_copy(acc_sh.at[pl.ds(s * (BACC // NT), BACC // NT)],
                        out_hbm.at[c, pl.ds(s * (BACC // NT), BACC // NT)])

    f = pl.kernel(
        body,
        out_type=jax.ShapeDtypeStruct((2, BACC, D), jnp.float32),
        mesh=_sc_mesh(),
        compiler_params=pltpu.CompilerParams(use_tc_tiling_on_sc=False),
        scratch_types=[
            pltpu.VMEM((PCH * CHUNK, D), jnp.float32),
            pltpu.VMEM((PCH, CHUNK), jnp.int32),
            pltpu.VMEM_SHARED((BACC, D), jnp.float32),
            pltpu.SemaphoreType.DMA,
        ],
    )
    return f(h5, batchp, zeros)


def _mm_body(x_ref, w_ref, o_ref):
    o_ref[...] = jnp.dot(x_ref[...], w_ref[...],
                         preferred_element_type=jnp.float32)


def _tc_y0(x4, w_bd):
    blk = 512
    return pl.pallas_call(
        _mm_body,
        grid=(pl.cdiv(N // 4, blk),),
        in_specs=[pl.BlockSpec((blk, 4 * 78), lambda i: (i, 0)),
                  pl.BlockSpec((4 * 78, 4 * D), lambda i: (0, 0))],
        out_specs=pl.BlockSpec((blk, 4 * D), lambda i: (i, 0)),
        out_shape=jax.ShapeDtypeStruct((N // 4, 4 * D), jnp.float32),
    )(x4, w_bd)


def _combine_body(y_ref, s_ref, wb_ref, wa_ref, vec_ref, o_ref):
    vec = vec_ref[...]
    sp = s_ref[...]
    t = jnp.maximum(y_ref[...] + sp[0] + sp[1] + vec[0:1, :], 0.0)
    z = jnp.maximum(jnp.dot(t, wb_ref[...],
                            preferred_element_type=jnp.float32)
                    + vec[1:2, :], 0.0)
    h = z * vec[2:3, :] + vec[3:4, :]
    o_ref[...] = jnp.dot(h, wa_ref[...], preferred_element_type=jnp.float32)


def _tc_combine(y, s_part, wb_bd, wa_bd, vecs4):
    blk = 1664
    dd = 4 * D
    return pl.pallas_call(
        _combine_body,
        grid=(pl.cdiv(N // 4, blk),),
        in_specs=[pl.BlockSpec((blk, dd), lambda i: (i, 0)),
                  pl.BlockSpec((2, blk, dd), lambda i: (0, i, 0)),
                  pl.BlockSpec((dd, dd), lambda i: (0, 0)),
                  pl.BlockSpec((dd, dd), lambda i: (0, 0)),
                  pl.BlockSpec((4, dd), lambda i: (0, 0))],
        out_specs=pl.BlockSpec((blk, dd), lambda i: (i, 0)),
        out_shape=jax.ShapeDtypeStruct((N // 4, dd), jnp.float32),
    )(y, s_part, wb_bd, wa_bd, vecs4)


def _final_layer_body(y_ref, s_ref, wb_ref, vec_ref, o_ref):
    vec = vec_ref[...]
    sp = s_ref[...]
    t = jnp.maximum(y_ref[...] + sp[0] + sp[1] + vec[0:1, :], 0.0)
    z = jnp.maximum(jnp.dot(t, wb_ref[...],
                            preferred_element_type=jnp.float32)
                    + vec[1:2, :], 0.0)
    o_ref[...] = z * vec[2:3, :] + vec[3:4, :]


def _tc_final_layer(y, s_part, wb_bd, vecs4):
    blk = 1664
    dd = 4 * D
    return pl.pallas_call(
        _final_layer_body,
        grid=(pl.cdiv(N // 4, blk),),
        in_specs=[pl.BlockSpec((blk, dd), lambda i: (i, 0)),
                  pl.BlockSpec((2, blk, dd), lambda i: (0, i, 0)),
                  pl.BlockSpec((dd, dd), lambda i: (0, 0)),
                  pl.BlockSpec((4, dd), lambda i: (0, 0))],
        out_specs=pl.BlockSpec((blk, dd), lambda i: (i, 0)),
        out_shape=jax.ShapeDtypeStruct((R // 4, dd), jnp.float32),
    )(y, s_part, wb_bd, vecs4)


def _q_body(t_ref, w_ref, o_ref):
    v = pl.program_id(0)
    mask = (t_ref[...] == v).astype(jnp.float32)
    o_ref[...] = jnp.dot(mask, w_ref[...],
                         preferred_element_type=jnp.float32)[None]


def _tc_q(target, wp):
    return pl.pallas_call(
        _q_body,
        grid=(V,),
        in_specs=[pl.BlockSpec((B, L), lambda v: (0, 0)),
                  pl.BlockSpec((L, 256), lambda v: (0, 0))],
        out_specs=pl.BlockSpec((1, B, 256), lambda v: (v, 0, 0)),
        out_shape=jax.ShapeDtypeStruct((V, B, 256), jnp.float32),
    )(target, wp)


def _conv2_body(q_ref, e_ref, cb_ref, o_ref):
    o_ref[...] = (jnp.dot(q_ref[...], e_ref[...],
                          preferred_element_type=jnp.float32)
                  + cb_ref[...])


def _tc_conv2(qbig, ehat, cbcol):
    blk = 2048
    return pl.pallas_call(
        _conv2_body,
        grid=(B * D // blk,),
        in_specs=[pl.BlockSpec((blk, V * 8), lambda i: (i, 0)),
                  pl.BlockSpec((V * 8, 121), lambda i: (0, 0)),
                  pl.BlockSpec((blk, 1), lambda i: (0, 0))],
        out_specs=pl.BlockSpec((blk, 121), lambda i: (i, 0)),
        out_shape=jax.ShapeDtypeStruct((B * D, 121), jnp.float32),
    )(qbig, ehat, cbcol)


def _xt_body(c_ref, w_ref, b_ref, o_ref):
    o_ref[...] = (jnp.dot(c_ref[...], w_ref[...],
                          preferred_element_type=jnp.float32)
                  + b_ref[...])


def _tc_xt(conv3, w2, bias_eff):
    return pl.pallas_call(
        _xt_body,
        out_shape=jax.ShapeDtypeStruct((B, 128), jnp.float32),
    )(conv3, w2, bias_eff)


def _head_body(p_ref, xt_ref, wxd_ref, bxd_ref, f1a_ref, f1b_ref, b1_ref,
               w2_ref, b2_ref, wo_ref, bo_ref, o_ref):
    pall = p_ref[...]
    pooled = pall[0, :B, :] + pall[1, :B, :]
    xd = jnp.maximum(jnp.dot(pooled, wxd_ref[...],
                             preferred_element_type=jnp.float32)
                     + bxd_ref[...], 0.0)
    xc = jnp.maximum(jnp.dot(xd, f1a_ref[...],
                             preferred_element_type=jnp.float32)
                     + jnp.dot(xt_ref[...], f1b_ref[...],
                               preferred_element_type=jnp.float32)
                     + b1_ref[...], 0.0)
    x2 = jnp.maximum(jnp.dot(xc, w2_ref[...],
                             preferred_element_type=jnp.float32)
                     + b2_ref[...], 0.0)
    o_ref[...] = (jnp.dot(x2, wo_ref[...],
                          preferred_element_type=jnp.float32)
                  + bo_ref[...])


def _tc_head(pooled_part, xt, p):
    args = (pooled_part, xt,
            p['fc1_xd_W'], p['fc1_xd_b'].reshape(1, 128),
            p['fc1_W'][:128], p['fc1_W'][128:], p['fc1_b'].reshape(1, 1024),
            p['fc2_W'], p['fc2_b'].reshape(1, 256),
            p['out_W'], p['out_b'].reshape(1, 1))
    return pl.pallas_call(
        _head_body,
        out_shape=jax.ShapeDtypeStruct((B, 1), jnp.float32),
    )(*args)


def kernel(x, edge_index, batch, target, params):
    p = params
    f32 = jnp.float32

    pad_i = jnp.arange(E_PAD - E, dtype=jnp.int32)
    srcp = jnp.concatenate([edge_index[0], pad_i % N]
                           ).reshape(NW, NSG, SGS, G * CHUNK)
    dstp = jnp.concatenate([edge_index[1], N + pad_i % (RACC - N)]
                           ).reshape(NW, NSG, SGS, G * CHUNK)
    pad_b = jnp.arange(R - N, dtype=jnp.int32)
    batchp = jnp.concatenate([batch, B + pad_b % (BACC - B)]
                             ).reshape(NW * PCH, CHUNK)
    zeros = jnp.zeros((RACC // NT, D), f32)

    eye4 = jnp.eye(4, dtype=f32)
    vecs4 = [jnp.stack([jnp.tile(p['b%da' % i], 4),
                        jnp.tile(p['b%db' % i], 4),
                        jnp.tile(p['bn%d_g' % i] / jnp.sqrt(f32(1.0 + 1e-5)),
                                 4),
                        jnp.tile(p['bn%d_b' % i], 4)]) for i in range(5)]
    wa_bd = [jnp.kron(eye4, p['W%da' % i]) for i in range(5)]
    wb_bd = [jnp.kron(eye4, p['W%db' % i]) for i in range(5)]

    x4 = x.reshape(N // 4, 4 * 78)
    y = _tc_y0(x4, jnp.kron(eye4, p['W0a']))
    s_part = _edge_segsum(y.reshape(N, D), srcp, dstp, zeros)

    wp = p['conv_W'].transpose(1, 0, 2).reshape(L, 256)
    q = _tc_q(target, wp)
    qbig = q.reshape(V, B, D, 8).transpose(1, 2, 0, 3).reshape(B * D, V * 8)
    ehat = jnp.stack([p['emb'][:, k:k + 121] for k in range(8)],
                     axis=1).reshape(V * 8, 121)
    cbcol = jnp.tile(p['conv_b'], 2048 // D).reshape(2048, 1)
    convflat = _tc_conv2(qbig, ehat, cbcol)
    conv3 = convflat.reshape(B, D * 121)
    xt = _tc_xt(conv3, p['fc1_xt_W'], p['fc1_xt_b'].reshape(1, 128))

    for i in range(4):
        y = _tc_combine(y, s_part.reshape(2, RACC // 4, 4 * D),
                        wb_bd[i], wa_bd[i + 1], vecs4[i])
        s_part = _edge_segsum(y.reshape(N, D), srcp, dstp, zeros)
    h5 = _tc_final_layer(y, s_part.reshape(2, RACC // 4, 4 * D),
                         wb_bd[4], vecs4[4])
    pooled_part = _pool_segsum(h5.reshape(R, D), batchp, zeros)

    return _tc_head(pooled_part, xt, p)
